# Initial kernel scaffold; baseline (speedup 1.0000x reference)
import jax
import jax.numpy as jnp
from jax.experimental import pallas as pl


def kernel(abundance_matrix, Y, W1, b1, W2, b2, W3, b3):
    raise NotImplementedError("write your pallas kernel here")


# speedup vs baseline: 1.4960x; 1.4960x over previous
import jax
import jax.numpy as jnp
from jax.experimental import pallas as pl
from jax.experimental.pallas import tpu as pltpu

_B1 = 1024
_B2 = 2048


def _stats_kernel(s_ref, y_ref, w1_ref, b1_ref, w2_ref, b2_ref, w3_ref, b3_ref,
                  m_out_ref, mx_ref, d_ref, n_ref):
    i = pl.program_id(0)
    nb = pl.num_programs(0)

    @pl.when(i == 0)
    def _init():
        mx_ref[...] = jnp.full_like(mx_ref, -1e30)
        d_ref[...] = jnp.zeros_like(d_ref)
        n_ref[...] = jnp.zeros_like(n_ref)

    s = s_ref[...]
    y = y_ref[...]
    h = jnp.maximum(y @ w1_ref[...] + b1_ref[...], 0.0)
    h = jnp.maximum(h @ w2_ref[...] + b2_ref[...], 0.0)
    logits = h @ w3_ref[...] + b3_ref[...]

    p = s.shape[1]
    c = jnp.argmax(s, axis=1)
    onehot = (c[:, None] == jax.lax.broadcasted_iota(jnp.int32, (1, p), 1)
              ).astype(jnp.float32)

    m_old = mx_ref[...]
    m_new = jnp.maximum(m_old, jnp.max(logits, axis=0, keepdims=True))
    scale = jnp.exp(m_old - m_new)
    e = jnp.exp(logits - m_new)
    d_blk = jax.lax.dot_general(onehot, e, (((0,), (0,)), ((), ())),
                                preferred_element_type=jnp.float32)
    n_blk = jax.lax.dot_general(onehot, e * y, (((0,), (0,)), ((), ())),
                                preferred_element_type=jnp.float32)
    mx_ref[...] = m_new
    d_ref[...] = d_ref[...] * scale + d_blk
    n_ref[...] = n_ref[...] * scale + n_blk

    @pl.when(i == nb - 1)
    def _finalize():
        dd = d_ref[...]
        m_out_ref[...] = jnp.where(
            dd > 0, n_ref[...] / jnp.maximum(dd, 1e-30), 0.0)


def _combine_kernel(s_ref, m_ref, out_ref):
    out_ref[...] = jnp.dot(s_ref[...], m_ref[...],
                           preferred_element_type=jnp.float32)


def kernel(abundance_matrix, Y, W1, b1, W2, b2, W3, b3):
    n, p = abundance_matrix.shape
    f = Y.shape[1]
    h = W1.shape[1]
    b1r = b1.reshape(1, h)
    b2r = b2.reshape(1, h)
    b3r = b3.reshape(1, f)

    nb1 = n // _B1
    M = pl.pallas_call(
        _stats_kernel,
        grid=(nb1,),
        in_specs=[
            pl.BlockSpec((_B1, p), lambda i: (i, 0)),
            pl.BlockSpec((_B1, f), lambda i: (i, 0)),
            pl.BlockSpec((W1.shape[0], h), lambda i: (0, 0)),
            pl.BlockSpec((1, h), lambda i: (0, 0)),
            pl.BlockSpec((h, h), lambda i: (0, 0)),
            pl.BlockSpec((1, h), lambda i: (0, 0)),
            pl.BlockSpec((h, f), lambda i: (0, 0)),
            pl.BlockSpec((1, f), lambda i: (0, 0)),
        ],
        out_specs=pl.BlockSpec((p, f), lambda i: (0, 0)),
        out_shape=jax.ShapeDtypeStruct((p, f), jnp.float32),
        scratch_shapes=[
            pltpu.VMEM((1, f), jnp.float32),
            pltpu.VMEM((p, f), jnp.float32),
            pltpu.VMEM((p, f), jnp.float32),
        ],
    )(abundance_matrix, Y, W1, b1r, W2, b2r, W3, b3r)

    nb2 = n // _B2
    Y_hat = pl.pallas_call(
        _combine_kernel,
        grid=(nb2,),
        in_specs=[
            pl.BlockSpec((_B2, p), lambda i: (i, 0)),
            pl.BlockSpec((p, f), lambda i: (0, 0)),
        ],
        out_specs=pl.BlockSpec((_B2, f), lambda i: (i, 0)),
        out_shape=jax.ShapeDtypeStruct((n, f), jnp.float32),
    )(abundance_matrix, M)
    return Y_hat


# speedup vs baseline: 1.7295x; 1.1560x over previous
import jax
import jax.numpy as jnp
from jax.experimental import pallas as pl
from jax.experimental.pallas import tpu as pltpu

_B1 = 2048
_B2 = 8192


def _stats_kernel(s_ref, y_ref, w1_ref, b1_ref, w2_ref, b2_ref, w3_ref, b3_ref,
                  m_out_ref, mx_ref, d_ref, n_ref):
    i = pl.program_id(0)
    nb = pl.num_programs(0)

    @pl.when(i == 0)
    def _init():
        mx_ref[...] = jnp.full_like(mx_ref, -1e30)
        d_ref[...] = jnp.zeros_like(d_ref)
        n_ref[...] = jnp.zeros_like(n_ref)

    s = s_ref[...]
    y = y_ref[...]
    h = jnp.maximum(y @ w1_ref[...] + b1_ref[...], 0.0)
    h = jnp.maximum(h @ w2_ref[...] + b2_ref[...], 0.0)
    logits = h @ w3_ref[...] + b3_ref[...]

    p = s.shape[1]
    c = jnp.argmax(s, axis=1)
    onehot = (c[:, None] == jax.lax.broadcasted_iota(jnp.int32, (1, p), 1)
              ).astype(jnp.float32)

    m_old = mx_ref[...]
    m_new = jnp.maximum(m_old, jnp.max(logits, axis=0, keepdims=True))
    scale = jnp.exp(m_old - m_new)
    e = jnp.exp(logits - m_new)
    d_blk = jax.lax.dot_general(onehot, e, (((0,), (0,)), ((), ())),
                                preferred_element_type=jnp.float32)
    n_blk = jax.lax.dot_general(onehot, e * y, (((0,), (0,)), ((), ())),
                                preferred_element_type=jnp.float32)
    mx_ref[...] = m_new
    d_ref[...] = d_ref[...] * scale + d_blk
    n_ref[...] = n_ref[...] * scale + n_blk

    @pl.when(i == nb - 1)
    def _finalize():
        dd = d_ref[...]
        m_out_ref[...] = jnp.where(
            dd > 0, n_ref[...] / jnp.maximum(dd, 1e-30), 0.0)


def _combine_kernel(s_ref, m_ref, out_ref):
    out_ref[...] = jnp.dot(s_ref[...], m_ref[...],
                           preferred_element_type=jnp.float32)


def kernel(abundance_matrix, Y, W1, b1, W2, b2, W3, b3):
    n, p = abundance_matrix.shape
    f = Y.shape[1]
    h = W1.shape[1]
    b1r = b1.reshape(1, h)
    b2r = b2.reshape(1, h)
    b3r = b3.reshape(1, f)

    nb1 = n // _B1
    M = pl.pallas_call(
        _stats_kernel,
        grid=(nb1,),
        in_specs=[
            pl.BlockSpec((_B1, p), lambda i: (i, 0)),
            pl.BlockSpec((_B1, f), lambda i: (i, 0)),
            pl.BlockSpec((W1.shape[0], h), lambda i: (0, 0)),
            pl.BlockSpec((1, h), lambda i: (0, 0)),
            pl.BlockSpec((h, h), lambda i: (0, 0)),
            pl.BlockSpec((1, h), lambda i: (0, 0)),
            pl.BlockSpec((h, f), lambda i: (0, 0)),
            pl.BlockSpec((1, f), lambda i: (0, 0)),
        ],
        out_specs=pl.BlockSpec((p, f), lambda i: (0, 0)),
        out_shape=jax.ShapeDtypeStruct((p, f), jnp.float32),
        scratch_shapes=[
            pltpu.VMEM((1, f), jnp.float32),
            pltpu.VMEM((p, f), jnp.float32),
            pltpu.VMEM((p, f), jnp.float32),
        ],
    )(abundance_matrix, Y, W1, b1r, W2, b2r, W3, b3r)

    nb2 = n // _B2
    Y_hat = pl.pallas_call(
        _combine_kernel,
        grid=(nb2,),
        in_specs=[
            pl.BlockSpec((_B2, p), lambda i: (i, 0)),
            pl.BlockSpec((p, f), lambda i: (0, 0)),
        ],
        out_specs=pl.BlockSpec((_B2, f), lambda i: (i, 0)),
        out_shape=jax.ShapeDtypeStruct((n, f), jnp.float32),
    )(abundance_matrix, M)
    return Y_hat
